# baseline (speedup 1.0000x reference)
import jax
import jax.numpy as jnp
from jax.experimental import pallas as pl


def _router_block(x_ref, wt_ref, b_ref, o_ref):
    logits = jnp.dot(x_ref[...].astype(jnp.bfloat16),
                     wt_ref[...].astype(jnp.bfloat16),
                     preferred_element_type=jnp.float32)
    logits = logits + b_ref[...]
    m = jnp.max(logits, axis=-1, keepdims=True)
    e = jnp.exp(logits - m)
    o_ref[...] = e / jnp.sum(e, axis=-1, keepdims=True)


def kernel(x, W, b):
    n_tokens, d_model = x.shape
    n_experts = W.shape[0]
    block_t = 512
    wt = W.T
    b2 = b.reshape(1, n_experts)
    return pl.pallas_call(
        _router_block,
        grid=(n_tokens // block_t,),
        in_specs=[
            pl.BlockSpec((block_t, d_model), lambda i: (i, 0)),
            pl.BlockSpec((d_model, n_experts), lambda i: (0, 0)),
            pl.BlockSpec((1, n_experts), lambda i: (0, 0)),
        ],
        out_specs=pl.BlockSpec((block_t, n_experts), lambda i: (i, 0)),
        out_shape=jax.ShapeDtypeStruct((n_tokens, n_experts), jnp.float32),
    )(x, wt, b2)


# speedup vs baseline: 1.0059x; 1.0059x over previous
import jax
import jax.numpy as jnp
from jax.experimental import pallas as pl
from jax.experimental.pallas import tpu as pltpu

_BLOCK_T = 512
_N_BUFS = 4


def _router_outer(x_hbm, wt_ref, b_ref, o_hbm):
    n_tokens, d_model = x_hbm.shape
    n_experts = o_hbm.shape[1]

    def body(x_blk, o_blk):
        logits = jnp.dot(x_blk[...], wt_ref[...],
                         preferred_element_type=jnp.float32)
        logits = logits + b_ref[...]
        m = jnp.max(logits, axis=-1, keepdims=True)
        e = jnp.exp(logits - m)
        o_blk[...] = e / jnp.sum(e, axis=-1, keepdims=True)

    pipe = pltpu.emit_pipeline(
        body,
        grid=(n_tokens // _BLOCK_T,),
        in_specs=[
            pl.BlockSpec((_BLOCK_T, d_model), lambda i: (i, 0),
                         pipeline_mode=pl.Buffered(buffer_count=_N_BUFS)),
        ],
        out_specs=[
            pl.BlockSpec((_BLOCK_T, n_experts), lambda i: (i, 0)),
        ],
    )
    pipe(x_hbm, o_hbm)


def kernel(x, W, b):
    n_tokens, d_model = x.shape
    n_experts = W.shape[0]
    wt = W.T
    b2 = b.reshape(1, n_experts)
    return pl.pallas_call(
        _router_outer,
        in_specs=[
            pl.BlockSpec(memory_space=pltpu.MemorySpace.HBM),
            pl.BlockSpec((d_model, n_experts), lambda: (0, 0)),
            pl.BlockSpec((1, n_experts), lambda: (0, 0)),
        ],
        out_specs=pl.BlockSpec(memory_space=pltpu.MemorySpace.HBM),
        out_shape=jax.ShapeDtypeStruct((n_tokens, n_experts), jnp.float32),
    )(x, wt, b2)


# speedup vs baseline: 1.0220x; 1.0160x over previous
import jax
import jax.numpy as jnp
from jax.experimental import pallas as pl
from jax.experimental.pallas import tpu as pltpu

_BLOCK_T = 512
_N_SPLIT = 4
_N_BUFS = 4


def _router_outer(x_hbm, wt_ref, b_ref, o_hbm):
    n_tokens, d_model = x_hbm.shape
    n_experts = o_hbm.shape[1]
    d_chunk = d_model // _N_SPLIT

    def body(*refs):
        x_chunks = refs[:_N_SPLIT]
        o_blk = refs[_N_SPLIT]
        logits = b_ref[...].astype(jnp.float32)
        acc = None
        for q in range(_N_SPLIT):
            part = jnp.dot(x_chunks[q][...],
                           wt_ref[pl.ds(q * d_chunk, d_chunk), :],
                           preferred_element_type=jnp.float32)
            acc = part if acc is None else acc + part
        logits = acc + logits
        m = jnp.max(logits, axis=-1, keepdims=True)
        e = jnp.exp(logits - m)
        o_blk[...] = e / jnp.sum(e, axis=-1, keepdims=True)

    in_specs = [
        pl.BlockSpec((_BLOCK_T, d_chunk),
                     lambda i, q=q: (i, q),
                     pipeline_mode=pl.Buffered(buffer_count=_N_BUFS))
        for q in range(_N_SPLIT)
    ]
    pipe = pltpu.emit_pipeline(
        body,
        grid=(n_tokens // _BLOCK_T,),
        in_specs=in_specs,
        out_specs=[
            pl.BlockSpec((_BLOCK_T, n_experts), lambda i: (i, 0)),
        ],
    )
    pipe(*([x_hbm] * _N_SPLIT), o_hbm)


def kernel(x, W, b):
    n_tokens, d_model = x.shape
    n_experts = W.shape[0]
    wt = W.T
    b2 = b.reshape(1, n_experts)
    return pl.pallas_call(
        _router_outer,
        in_specs=[
            pl.BlockSpec(memory_space=pltpu.MemorySpace.HBM),
            pl.BlockSpec((d_model, n_experts), lambda: (0, 0)),
            pl.BlockSpec((1, n_experts), lambda: (0, 0)),
        ],
        out_specs=pl.BlockSpec(memory_space=pltpu.MemorySpace.HBM),
        out_shape=jax.ShapeDtypeStruct((n_tokens, n_experts), jnp.float32),
    )(x, wt, b2)
